# baseline (speedup 1.0000x reference)
import jax
import jax.numpy as jnp
from jax import lax
from jax.experimental import pallas as pl
from jax.experimental.pallas import tpu as pltpu
from jax.experimental.pallas import tpu_sc as plsc

N_FIELDS = 26
BATCH = 16384
EMBED_DIM = 32
NC = 2
NS = 16
NW = NC * NS
B_PER_W = BATCH // NW
CH = 128
NCHUNK = B_PER_W // CH

_mesh = plsc.VectorSubcoreMesh(
    core_axis_name="c", subcore_axis_name="s", num_cores=NC, num_subcores=NS
)


def _sc_body(*refs):
    fs = refs[:N_FIELDS]
    Ws = refs[N_FIELDS:2 * N_FIELDS]
    out = refs[2 * N_FIELDS]
    idx, acc, b0, b1, sem_i, sem_a, s0, s1 = refs[2 * N_FIELDS + 1:]
    bufs = (b0, b1)
    sems = (s0, s1)

    wid = lax.axis_index("s") * NC + lax.axis_index("c")
    base = wid * B_PER_W

    def chunk_body(c, carry):
        row0 = base + c * CH
        iw = [
            pltpu.async_copy(fs[f].at[pl.ds(row0, CH)], idx.at[f], sem_i)
            for f in range(N_FIELDS)
        ]
        for w in iw:
            w.wait()
        cps = {}
        cps[0] = pltpu.async_copy(Ws[0].at[idx.at[0]], acc, sem_a)
        cps[1] = pltpu.async_copy(Ws[1].at[idx.at[1]], bufs[0], sems[0])
        cps[2] = pltpu.async_copy(Ws[2].at[idx.at[2]], bufs[1], sems[1])
        cps[0].wait()
        for f in range(1, N_FIELDS):
            cps[f].wait()
            bb = bufs[(f - 1) % 2]

            @plsc.parallel_loop(0, CH, 1, unroll=8)
            def _accum(r, bb=bb):
                plsc.addupdate(acc.at[r, pl.ds(0, 16)], bb[r, pl.ds(0, 16)])
                plsc.addupdate(acc.at[r, pl.ds(16, 16)], bb[r, pl.ds(16, 16)])

            if f + 2 < N_FIELDS:
                cps[f + 2] = pltpu.async_copy(
                    Ws[f + 2].at[idx.at[f + 2]], bufs[(f + 1) % 2], sems[(f + 1) % 2]
                )
        pltpu.sync_copy(acc, out.at[pl.ds(row0, CH)])
        return carry

    lax.fori_loop(0, NCHUNK, chunk_body, 0)


def kernel(f0, f1, f2, f3, f4, f5, f6, f7, f8, f9, f10, f11, f12, f13, f14, f15, f16, f17, f18, f19, f20, f21, f22, f23, f24, f25, W0, W1, W2, W3, W4, W5, W6, W7, W8, W9, W10, W11, W12, W13, W14, W15, W16, W17, W18, W19, W20, W21, W22, W23, W24, W25):
    fs = [f0, f1, f2, f3, f4, f5, f6, f7, f8, f9, f10, f11, f12,
          f13, f14, f15, f16, f17, f18, f19, f20, f21, f22, f23, f24, f25]
    Ws = [W0, W1, W2, W3, W4, W5, W6, W7, W8, W9, W10, W11, W12,
          W13, W14, W15, W16, W17, W18, W19, W20, W21, W22, W23, W24, W25]
    fs = [f.astype(jnp.int32) for f in fs]
    run = pl.kernel(
        _sc_body,
        out_type=jax.ShapeDtypeStruct((BATCH, EMBED_DIM), jnp.float32),
        mesh=_mesh,
        compiler_params=pltpu.CompilerParams(use_tc_tiling_on_sc=False),
        scratch_types=[
            pltpu.VMEM((N_FIELDS, CH), jnp.int32),
            pltpu.VMEM((CH, EMBED_DIM), jnp.float32),
            pltpu.VMEM((CH, EMBED_DIM), jnp.float32),
            pltpu.VMEM((CH, EMBED_DIM), jnp.float32),
            pltpu.SemaphoreType.DMA,
            pltpu.SemaphoreType.DMA,
            pltpu.SemaphoreType.DMA,
            pltpu.SemaphoreType.DMA,
        ],
    )
    return run(*fs, *Ws)


# speedup vs baseline: 1.4540x; 1.4540x over previous
import jax
import jax.numpy as jnp
from jax import lax
from jax.experimental import pallas as pl
from jax.experimental.pallas import tpu as pltpu
from jax.experimental.pallas import tpu_sc as plsc

N_FIELDS = 26
BATCH = 16384
EMBED_DIM = 32
VOCAB_ROWS = 100005
N_GROUPS = 7
PACK_CHUNK = 512
NPAD = 196 * PACK_CHUNK
NC = 2
NS = 16
NW = NC * NS
B_PER_W = BATCH // NW
CH = 64
NCHUNK = B_PER_W // CH

_mesh = plsc.VectorSubcoreMesh(
    core_axis_name="c", subcore_axis_name="s", num_cores=NC, num_subcores=NS
)


def _pack_body(*refs):
    ins = refs[:N_FIELDS]
    outs = refs[N_FIELDS:]
    for g in range(N_GROUPS):
        parts = []
        for k in range(4):
            f = 4 * g + k
            if f < N_FIELDS:
                parts.append(ins[f][...].T)
            else:
                parts.append(jnp.zeros((PACK_CHUNK, EMBED_DIM), jnp.float32))
        outs[g][...] = jnp.concatenate(parts, axis=1)


_pack = pl.pallas_call(
    _pack_body,
    grid=(NPAD // PACK_CHUNK,),
    in_specs=[
        pl.BlockSpec((EMBED_DIM, PACK_CHUNK), lambda j: (0, j))
        for _ in range(N_FIELDS)
    ],
    out_specs=[
        pl.BlockSpec((PACK_CHUNK, 128), lambda j: (j, 0))
        for _ in range(N_GROUPS)
    ],
    out_shape=[
        jax.ShapeDtypeStruct((NPAD, 128), jnp.float32) for _ in range(N_GROUPS)
    ],
)


def _sc_body(*refs):
    fs = refs[:N_FIELDS]
    Gs = refs[N_FIELDS:N_FIELDS + N_GROUPS]
    out = refs[N_FIELDS + N_GROUPS]
    idx, accA, accB, outbuf, sem_i, sem_0, sem_a = refs[N_FIELDS + N_GROUPS + 1:]
    accs = (accA, accB)

    wid = lax.axis_index("s") * NC + lax.axis_index("c")
    base = wid * B_PER_W

    iw = [
        pltpu.async_copy(fs[f].at[pl.ds(base, B_PER_W)], idx.at[f], sem_i)
        for f in range(N_FIELDS)
    ]
    for w in iw:
        w.wait()

    def issue_init(c, acc):
        sl = pl.ds(c * CH, CH)
        return [
            pltpu.async_copy(Gs[f // 4].at[idx.at[f, sl]], acc.at[f % 4], sem_0)
            for f in range(4)
        ]

    pend_init = issue_init(0, accs[0])
    for c in range(NCHUNK):
        p = c % 2
        acc = accs[p]
        sl = pl.ds(c * CH, CH)
        for w in pend_init:
            w.wait()
        ga = [
            pltpu.async_copy(
                Gs[f // 4].at[idx.at[f, sl]], acc.at[f % 4], sem_a, add=True
            )
            for f in range(4, N_FIELDS)
        ]
        if c + 1 < NCHUNK:
            pend_init = issue_init(c + 1, accs[1 - p])
        for w in ga:
            w.wait()

        @plsc.parallel_loop(0, CH, step=1, unroll=8)
        def _row(r):
            for k in range(2):
                v = acc[0, r, pl.ds(16 * k, 16)]
                for q in range(1, 4):
                    v = v + acc[q, r, pl.ds(32 * q + 16 * k, 16)]
                outbuf[r, pl.ds(16 * k, 16)] = v

        pltpu.sync_copy(outbuf, out.at[pl.ds(base + c * CH, CH)])


_gather = pl.kernel(
    _sc_body,
    out_type=jax.ShapeDtypeStruct((BATCH, 128), jnp.float32),
    mesh=_mesh,
    scratch_types=[
        pltpu.VMEM((N_FIELDS, B_PER_W), jnp.int32),
        pltpu.VMEM((4, CH, 128), jnp.float32),
        pltpu.VMEM((4, CH, 128), jnp.float32),
        pltpu.VMEM((CH, 128), jnp.float32),
        pltpu.SemaphoreType.DMA,
        pltpu.SemaphoreType.DMA,
        pltpu.SemaphoreType.DMA,
    ],
)


def kernel(f0, f1, f2, f3, f4, f5, f6, f7, f8, f9, f10, f11, f12, f13, f14, f15, f16, f17, f18, f19, f20, f21, f22, f23, f24, f25, W0, W1, W2, W3, W4, W5, W6, W7, W8, W9, W10, W11, W12, W13, W14, W15, W16, W17, W18, W19, W20, W21, W22, W23, W24, W25):
    fs = [f0, f1, f2, f3, f4, f5, f6, f7, f8, f9, f10, f11, f12,
          f13, f14, f15, f16, f17, f18, f19, f20, f21, f22, f23, f24, f25]
    Ws = [W0, W1, W2, W3, W4, W5, W6, W7, W8, W9, W10, W11, W12,
          W13, W14, W15, W16, W17, W18, W19, W20, W21, W22, W23, W24, W25]
    fs = [f.astype(jnp.int32) for f in fs]
    groups = _pack(*[W.T for W in Ws])
    out128 = _gather(*fs, *groups)
    return out128[:, :EMBED_DIM]


# speedup vs baseline: 1.5197x; 1.0452x over previous
import jax
import jax.numpy as jnp
from jax import lax
from jax.experimental import pallas as pl
from jax.experimental.pallas import tpu as pltpu
from jax.experimental.pallas import tpu_sc as plsc

N_FIELDS = 26
BATCH = 16384
EMBED_DIM = 32
VOCAB_ROWS = 100005
N_GROUPS = 7
PACK_CHUNK = 1024
NPAD = 98 * PACK_CHUNK
NC = 2
NS = 16
NW = NC * NS
B_PER_W = BATCH // NW
CH = 64
NCHUNK = B_PER_W // CH

_mesh = plsc.VectorSubcoreMesh(
    core_axis_name="c", subcore_axis_name="s", num_cores=NC, num_subcores=NS
)


def _pack_body(*refs):
    ins = refs[:N_FIELDS]
    outs = refs[N_FIELDS:]
    for g in range(N_GROUPS):
        parts = []
        for k in range(4):
            f = 4 * g + k
            if f < N_FIELDS:
                parts.append(ins[f][...].T)
            else:
                parts.append(jnp.zeros((PACK_CHUNK, EMBED_DIM), jnp.float32))
        outs[g][...] = jnp.concatenate(parts, axis=1)


_pack = pl.pallas_call(
    _pack_body,
    grid=(NPAD // PACK_CHUNK,),
    in_specs=[
        pl.BlockSpec((EMBED_DIM, PACK_CHUNK), lambda j: (0, j))
        for _ in range(N_FIELDS)
    ],
    out_specs=[
        pl.BlockSpec((PACK_CHUNK, 128), lambda j: (j, 0))
        for _ in range(N_GROUPS)
    ],
    out_shape=[
        jax.ShapeDtypeStruct((NPAD, 128), jnp.float32) for _ in range(N_GROUPS)
    ],
)


def _sc_body(*refs):
    fs = refs[:N_FIELDS]
    Gs = refs[N_FIELDS:N_FIELDS + N_GROUPS]
    out = refs[N_FIELDS + N_GROUPS]
    idx, accA, accB, outbuf, sem_i, sem_0, sem_a = refs[N_FIELDS + N_GROUPS + 1:]
    accs = (accA, accB)

    wid = lax.axis_index("s") * NC + lax.axis_index("c")
    base = wid * B_PER_W

    iw = [
        pltpu.async_copy(fs[f].at[pl.ds(base, B_PER_W)], idx.at[f], sem_i)
        for f in range(N_FIELDS)
    ]
    for w in iw:
        w.wait()

    def issue_init(c, acc):
        sl = pl.ds(c * CH, CH)
        return [
            pltpu.async_copy(Gs[f // 4].at[idx.at[f, sl]], acc.at[f % 4], sem_0)
            for f in range(4)
        ]

    pend_init = issue_init(0, accs[0])
    for c in range(NCHUNK):
        p = c % 2
        acc = accs[p]
        sl = pl.ds(c * CH, CH)
        for w in pend_init:
            w.wait()
        ga = [
            pltpu.async_copy(
                Gs[f // 4].at[idx.at[f, sl]], acc.at[f % 4], sem_a, add=True
            )
            for f in range(4, N_FIELDS)
        ]
        if c + 1 < NCHUNK:
            pend_init = issue_init(c + 1, accs[1 - p])
        for w in ga:
            w.wait()

        @plsc.parallel_loop(0, CH, step=1, unroll=8)
        def _row(r):
            for k in range(2):
                v = acc[0, r, pl.ds(16 * k, 16)]
                for q in range(1, 4):
                    v = v + acc[q, r, pl.ds(32 * q + 16 * k, 16)]
                outbuf[r, pl.ds(16 * k, 16)] = v

        pltpu.sync_copy(outbuf, out.at[pl.ds(base + c * CH, CH)])


_gather = pl.kernel(
    _sc_body,
    out_type=jax.ShapeDtypeStruct((BATCH, 128), jnp.float32),
    mesh=_mesh,
    scratch_types=[
        pltpu.VMEM((N_FIELDS, B_PER_W), jnp.int32),
        pltpu.VMEM((4, CH, 128), jnp.float32),
        pltpu.VMEM((4, CH, 128), jnp.float32),
        pltpu.VMEM((CH, 128), jnp.float32),
        pltpu.SemaphoreType.DMA,
        pltpu.SemaphoreType.DMA,
        pltpu.SemaphoreType.DMA,
    ],
)


def kernel(f0, f1, f2, f3, f4, f5, f6, f7, f8, f9, f10, f11, f12, f13, f14, f15, f16, f17, f18, f19, f20, f21, f22, f23, f24, f25, W0, W1, W2, W3, W4, W5, W6, W7, W8, W9, W10, W11, W12, W13, W14, W15, W16, W17, W18, W19, W20, W21, W22, W23, W24, W25):
    fs = [f0, f1, f2, f3, f4, f5, f6, f7, f8, f9, f10, f11, f12,
          f13, f14, f15, f16, f17, f18, f19, f20, f21, f22, f23, f24, f25]
    Ws = [W0, W1, W2, W3, W4, W5, W6, W7, W8, W9, W10, W11, W12,
          W13, W14, W15, W16, W17, W18, W19, W20, W21, W22, W23, W24, W25]
    fs = [f.astype(jnp.int32) for f in fs]
    groups = _pack(*[W.T for W in Ws])
    out128 = _gather(*fs, *groups)
    return out128[:, :EMBED_DIM]


# speedup vs baseline: 1.5580x; 1.0252x over previous
import jax
import jax.numpy as jnp
from jax import lax
from jax.experimental import pallas as pl
from jax.experimental.pallas import tpu as pltpu
from jax.experimental.pallas import tpu_sc as plsc

N_FIELDS = 26
BATCH = 16384
EMBED_DIM = 32
VOCAB_ROWS = 100005
N_GROUPS = 7
PACK_CHUNK = 2048
NPAD = 49 * PACK_CHUNK
NC = 2
NS = 16
NW = NC * NS
B_PER_W = BATCH // NW
CH = 64
NCHUNK = B_PER_W // CH

_mesh = plsc.VectorSubcoreMesh(
    core_axis_name="c", subcore_axis_name="s", num_cores=NC, num_subcores=NS
)


def _pack_body(*refs):
    ins = refs[:N_FIELDS]
    outs = refs[N_FIELDS:]
    for g in range(N_GROUPS):
        parts = []
        for k in range(4):
            f = 4 * g + k
            if f < N_FIELDS:
                parts.append(ins[f][...].T)
            else:
                parts.append(jnp.zeros((PACK_CHUNK, EMBED_DIM), jnp.float32))
        outs[g][...] = jnp.concatenate(parts, axis=1)


_pack = pl.pallas_call(
    _pack_body,
    grid=(NPAD // PACK_CHUNK,),
    in_specs=[
        pl.BlockSpec((EMBED_DIM, PACK_CHUNK), lambda j: (0, j))
        for _ in range(N_FIELDS)
    ],
    out_specs=[
        pl.BlockSpec((PACK_CHUNK, 128), lambda j: (j, 0))
        for _ in range(N_GROUPS)
    ],
    out_shape=[
        jax.ShapeDtypeStruct((NPAD, 128), jnp.float32) for _ in range(N_GROUPS)
    ],
)


def _sc_body(*refs):
    fs = refs[:N_FIELDS]
    Gs = refs[N_FIELDS:N_FIELDS + N_GROUPS]
    out = refs[N_FIELDS + N_GROUPS]
    idx, accA, accB, outbuf, sem_i, sem_0, sem_a = refs[N_FIELDS + N_GROUPS + 1:]
    accs = (accA, accB)

    wid = lax.axis_index("s") * NC + lax.axis_index("c")
    base = wid * B_PER_W

    iw = [
        pltpu.async_copy(fs[f].at[pl.ds(base, B_PER_W)], idx.at[f], sem_i)
        for f in range(N_FIELDS)
    ]
    for w in iw:
        w.wait()

    def issue_init(c, acc):
        sl = pl.ds(c * CH, CH)
        return [
            pltpu.async_copy(Gs[f // 4].at[idx.at[f, sl]], acc.at[f % 4], sem_0)
            for f in range(4)
        ]

    pend_init = issue_init(0, accs[0])
    for c in range(NCHUNK):
        p = c % 2
        acc = accs[p]
        sl = pl.ds(c * CH, CH)
        for w in pend_init:
            w.wait()
        ga = [
            pltpu.async_copy(
                Gs[f // 4].at[idx.at[f, sl]], acc.at[f % 4], sem_a, add=True
            )
            for f in range(4, N_FIELDS)
        ]
        if c + 1 < NCHUNK:
            pend_init = issue_init(c + 1, accs[1 - p])
        for w in ga:
            w.wait()

        @plsc.parallel_loop(0, CH, step=1, unroll=8)
        def _row(r):
            for k in range(2):
                v = acc[0, r, pl.ds(16 * k, 16)]
                for q in range(1, 4):
                    v = v + acc[q, r, pl.ds(32 * q + 16 * k, 16)]
                outbuf[r, pl.ds(16 * k, 16)] = v

        pltpu.sync_copy(outbuf, out.at[pl.ds(base + c * CH, CH)])


_gather = pl.kernel(
    _sc_body,
    out_type=jax.ShapeDtypeStruct((BATCH, 128), jnp.float32),
    mesh=_mesh,
    scratch_types=[
        pltpu.VMEM((N_FIELDS, B_PER_W), jnp.int32),
        pltpu.VMEM((4, CH, 128), jnp.float32),
        pltpu.VMEM((4, CH, 128), jnp.float32),
        pltpu.VMEM((CH, 128), jnp.float32),
        pltpu.SemaphoreType.DMA,
        pltpu.SemaphoreType.DMA,
        pltpu.SemaphoreType.DMA,
    ],
)


def kernel(f0, f1, f2, f3, f4, f5, f6, f7, f8, f9, f10, f11, f12, f13, f14, f15, f16, f17, f18, f19, f20, f21, f22, f23, f24, f25, W0, W1, W2, W3, W4, W5, W6, W7, W8, W9, W10, W11, W12, W13, W14, W15, W16, W17, W18, W19, W20, W21, W22, W23, W24, W25):
    fs = [f0, f1, f2, f3, f4, f5, f6, f7, f8, f9, f10, f11, f12,
          f13, f14, f15, f16, f17, f18, f19, f20, f21, f22, f23, f24, f25]
    Ws = [W0, W1, W2, W3, W4, W5, W6, W7, W8, W9, W10, W11, W12,
          W13, W14, W15, W16, W17, W18, W19, W20, W21, W22, W23, W24, W25]
    fs = [f.astype(jnp.int32) for f in fs]
    groups = _pack(*[W.T for W in Ws])
    out128 = _gather(*fs, *groups)
    return out128[:, :EMBED_DIM]
